# baseline (speedup 1.0000x reference)
import functools

import jax
import jax.numpy as jnp
from jax import lax
from jax.experimental import pallas as pl
from jax.experimental.pallas import tpu as pltpu

B = 2048
SEG = 1024
LAT = 64
K = 512
EMB = 64

BM = 1024

_NT = (((1,), (1,)), ((), ()))
_NN = (((1,), (0,)), ((), ()))


def _vqvae_kernel(x_ref, w1_ref, b1_ref, w2_ref, b2_ref, w3_ref, b3_ref,
                  w4_ref, b4_ref, cb_ref, cbt_ref, dw1_ref, db1_ref, dw2_ref,
                  db2_ref, dw3_ref, db3_ref, dw4_ref, db4_ref,
                  xr_ref, z_ref, zq_ref,
                  w1_v, w2_v, w3_v, w4_v, dw1_v, dw2_v, dw3_v, dw4_v,
                  s1, s2, s3, s4, s5, s6, s7, s8):
    f32 = jnp.float32
    is0 = pl.program_id(0) == 0

    copies = [
        pltpu.make_async_copy(w1_ref, w1_v, s1),
        pltpu.make_async_copy(w2_ref, w2_v, s2),
        pltpu.make_async_copy(w3_ref, w3_v, s3),
        pltpu.make_async_copy(w4_ref, w4_v, s4),
        pltpu.make_async_copy(dw1_ref, dw1_v, s5),
        pltpu.make_async_copy(dw2_ref, dw2_v, s6),
        pltpu.make_async_copy(dw3_ref, dw3_v, s7),
        pltpu.make_async_copy(dw4_ref, dw4_v, s8),
    ]

    @pl.when(is0)
    def _start():
        for c in copies:
            c.start()

    def dense(h, w_ref, b_ref, copy=None, relu=True):
        if copy is not None:
            @pl.when(is0)
            def _wait():
                copy.wait()
        o = lax.dot_general(h, w_ref[...], _NT,
                            preferred_element_type=f32) + b_ref[...]
        return jnp.maximum(o, 0.0) if relu else o

    h = dense(x_ref[...], w1_v, b1_ref, copies[0])
    h = dense(h, w2_v, b2_ref, copies[1])
    h = dense(h, w3_v, b3_ref, copies[2])
    z = dense(h, w4_v, b4_ref, copies[3], relu=False)
    z_ref[...] = z

    cb = cb_ref[...]
    cbt = cbt_ref[...]

    def split3(v):
        bb = lax.bitcast_convert_type(v, jnp.uint32)
        v0 = lax.bitcast_convert_type(bb & jnp.uint32(0xFFFF0000), f32)
        r = v - v0
        rb = lax.bitcast_convert_type(r, jnp.uint32)
        v1 = lax.bitcast_convert_type(rb & jnp.uint32(0xFFFF0000), f32)
        return v0, v1, r - v1

    z0, z1, z2 = split3(z)
    c0, c1, c2 = split3(-cbt)
    ones = jnp.ones((BM, 8), f32)
    cn2 = 0.5 * jnp.sum(cbt * cbt, axis=0, keepdims=True)
    n0, n1, n2 = split3(cn2)
    cn2rows = jnp.concatenate([n0, n1, n2, jnp.zeros((5, K), f32)], axis=0)
    zs = jnp.concatenate([z0, z0, z1, z0, z2, z1, ones], axis=1)
    cs = jnp.concatenate([c0, c1, c0, c2, c0, c1, cn2rows], axis=0)
    dist = lax.dot_general(zs, cs, _NN, preferred_element_type=f32)
    idx = jnp.argmin(dist, axis=1)[:, None]
    iota = lax.broadcasted_iota(jnp.int32, (BM, K), 1)
    onehot = (iota == idx).astype(f32)
    z_q = lax.dot_general(onehot, cb, _NN, preferred_element_type=f32)
    zq_ref[...] = z_q

    h = dense(z_q, dw1_v, db1_ref, copies[4])
    h = dense(h, dw2_v, db2_ref, copies[5])
    h = dense(h, dw3_v, db3_ref, copies[6])
    xr_ref[...] = dense(h, dw4_v, db4_ref, copies[7], relu=False)


@functools.partial(jax.jit, static_argnames=())
def kernel(x, enc_w1, enc_b1, enc_w2, enc_b2, enc_w3, enc_b3, enc_w4, enc_b4,
           codebook, dec_w1, dec_b1, dec_w2, dec_b2, dec_w3, dec_b3, dec_w4,
           dec_b4):
    def full(a):
        return pl.BlockSpec(a.shape, lambda i: (0,) * a.ndim)

    def hbm():
        return pl.BlockSpec(memory_space=pl.ANY)

    def rowblk(cols):
        return pl.BlockSpec((BM, cols), lambda i: (i, 0))

    biases2d = [b.reshape(1, -1) for b in
                (enc_b1, enc_b2, enc_b3, enc_b4, dec_b1, dec_b2, dec_b3,
                 dec_b4)]
    cbt = codebook.T

    wshapes = [enc_w1.shape, enc_w2.shape, enc_w3.shape, enc_w4.shape,
               dec_w1.shape, dec_w2.shape, dec_w3.shape, dec_w4.shape]
    grid = (B // BM,)
    out_shape = (
        jax.ShapeDtypeStruct((B, SEG), jnp.float32),
        jax.ShapeDtypeStruct((B, LAT), jnp.float32),
        jax.ShapeDtypeStruct((B, LAT), jnp.float32),
    )
    xr, z, zq = pl.pallas_call(
        _vqvae_kernel,
        grid=grid,
        in_specs=[
            rowblk(SEG),
            hbm(), full(biases2d[0]),
            hbm(), full(biases2d[1]),
            hbm(), full(biases2d[2]),
            hbm(), full(biases2d[3]),
            full(codebook), full(cbt),
            hbm(), full(biases2d[4]),
            hbm(), full(biases2d[5]),
            hbm(), full(biases2d[6]),
            hbm(), full(biases2d[7]),
        ],
        out_specs=(rowblk(SEG), rowblk(LAT), rowblk(LAT)),
        out_shape=out_shape,
        scratch_shapes=(
            [pltpu.VMEM(s, jnp.float32) for s in wshapes]
            + [pltpu.SemaphoreType.DMA] * 8
        ),
    )(x, enc_w1, biases2d[0], enc_w2, biases2d[1], enc_w3, biases2d[2],
      enc_w4, biases2d[3], codebook, cbt, dec_w1, biases2d[4], dec_w2,
      biases2d[5], dec_w3, biases2d[6], dec_w4, biases2d[7])
    return (xr, z, zq)


# speedup vs baseline: 1.0749x; 1.0749x over previous
import functools

import jax
import jax.numpy as jnp
from jax import lax
from jax.experimental import pallas as pl

B = 2048
SEG = 1024
LAT = 64
K = 512
EMB = 64

BM = 1024

_NT = (((1,), (1,)), ((), ()))
_NN = (((1,), (0,)), ((), ()))


def _vqvae_kernel(x_ref, w1_ref, b1_ref, w2_ref, b2_ref, w3_ref, b3_ref,
                  w4_ref, b4_ref, cb_ref, cbt_ref, dw1_ref, db1_ref, dw2_ref,
                  db2_ref, dw3_ref, db3_ref, dw4_ref, db4_ref,
                  xr_ref, z_ref, zq_ref):
    f32 = jnp.float32

    def dense(h, w_ref, b_ref, relu=True, precision=None):
        o = lax.dot_general(h, w_ref[...], _NT, preferred_element_type=f32,
                            precision=precision) + b_ref[...]
        return jnp.maximum(o, 0.0) if relu else o

    h = dense(x_ref[...], w1_ref, b1_ref)
    h = dense(h, w2_ref, b2_ref)
    h = dense(h, w3_ref, b3_ref)
    z = dense(h, w4_ref, b4_ref, relu=False)
    z_ref[...] = z

    cb = cb_ref[...]
    cbt = cbt_ref[...]

    def split3(v):
        b = lax.bitcast_convert_type(v, jnp.uint32)
        v0 = lax.bitcast_convert_type(b & jnp.uint32(0xFFFF0000), f32)
        r = v - v0
        rb = lax.bitcast_convert_type(r, jnp.uint32)
        v1 = lax.bitcast_convert_type(rb & jnp.uint32(0xFFFF0000), f32)
        return v0, v1, r - v1

    z0, z1, z2 = split3(z)
    c0, c1, c2 = split3(-cbt)
    ones = jnp.ones((BM, 8), f32)
    cn2 = 0.5 * jnp.sum(cbt * cbt, axis=0, keepdims=True)
    n0, n1, n2 = split3(cn2)
    cn2rows = jnp.concatenate([n0, n1, n2, jnp.zeros((5, K), f32)], axis=0)
    zs = jnp.concatenate([z0, z0, z1, z0, z2, z1, ones], axis=1)
    cs = jnp.concatenate([c0, c1, c0, c2, c0, c1, cn2rows], axis=0)
    dist = lax.dot_general(zs, cs, _NN, preferred_element_type=f32)
    idx = jnp.argmin(dist, axis=1)[:, None]
    iota = lax.broadcasted_iota(jnp.int32, (BM, K), 1)
    onehot = (iota == idx).astype(f32)
    z_q = lax.dot_general(onehot, cb, _NN, preferred_element_type=f32)
    zq_ref[...] = z_q

    h = dense(z_q, dw1_ref, db1_ref)
    h = dense(h, dw2_ref, db2_ref)
    h = dense(h, dw3_ref, db3_ref)
    xr_ref[...] = dense(h, dw4_ref, db4_ref, relu=False)


@functools.partial(jax.jit, static_argnames=())
def kernel(x, enc_w1, enc_b1, enc_w2, enc_b2, enc_w3, enc_b3, enc_w4, enc_b4,
           codebook, dec_w1, dec_b1, dec_w2, dec_b2, dec_w3, dec_b3, dec_w4,
           dec_b4):
    def full(a):
        return pl.BlockSpec(a.shape, lambda i: (0,) * a.ndim)

    def rowblk(cols):
        return pl.BlockSpec((BM, cols), lambda i: (i, 0))

    biases2d = [b.reshape(1, -1) for b in
                (enc_b1, enc_b2, enc_b3, enc_b4, dec_b1, dec_b2, dec_b3,
                 dec_b4)]
    cbt = codebook.T

    grid = (B // BM,)
    out_shape = (
        jax.ShapeDtypeStruct((B, SEG), jnp.float32),
        jax.ShapeDtypeStruct((B, LAT), jnp.float32),
        jax.ShapeDtypeStruct((B, LAT), jnp.float32),
    )
    xr, z, zq = pl.pallas_call(
        _vqvae_kernel,
        grid=grid,
        in_specs=[
            rowblk(SEG),
            full(enc_w1), full(biases2d[0]),
            full(enc_w2), full(biases2d[1]),
            full(enc_w3), full(biases2d[2]),
            full(enc_w4), full(biases2d[3]),
            full(codebook), full(cbt),
            full(dec_w1), full(biases2d[4]),
            full(dec_w2), full(biases2d[5]),
            full(dec_w3), full(biases2d[6]),
            full(dec_w4), full(biases2d[7]),
        ],
        out_specs=(rowblk(SEG), rowblk(LAT), rowblk(LAT)),
        out_shape=out_shape,
    )(x, enc_w1, biases2d[0], enc_w2, biases2d[1], enc_w3, biases2d[2],
      enc_w4, biases2d[3], codebook, cbt, dec_w1, biases2d[4], dec_w2,
      biases2d[5],
      dec_w3, biases2d[6], dec_w4, biases2d[7])
    return (xr, z, zq)
